# Initial kernel scaffold; baseline (speedup 1.0000x reference)
import jax
import jax.numpy as jnp
from jax import lax
from jax.experimental import pallas as pl
from jax.experimental.pallas import tpu as pltpu
from jax.experimental.pallas import tpu_sc as plsc

N = 100000
D = 128
MAX_DEG = 10
PER_DEG = 9090
DEG0 = N - MAX_DEG * PER_DEG
STARTS = [0, DEG0] + [DEG0 + PER_DEG * k for k in range(1, MAX_DEG)]
SIZES = [DEG0] + [PER_DEG] * MAX_DEG

C = 64
NW = 32
LANES = 16


def _body(atoms, a1, a2, a3, a4, a5, a6, a7, a8, a9, a10, out,
          i1, i2, i3, i4, i5, i6, i7, i8, i9, i10,
          self_v, g_v, out_v, sem):
    adjs = [a1, a2, a3, a4, a5, a6, a7, a8, a9, a10]
    idxs = [i1, i2, i3, i4, i5, i6, i7, i8, i9, i10]
    wid = lax.axis_index("s") * 2 + lax.axis_index("c")

    for deg in range(0, MAX_DEG + 1):
        base = STARTS[deg]
        n_rows = SIZES[deg]
        n_chunks = (n_rows + C - 1) // C
        last_r0 = n_rows - C
        k_per_w = (n_chunks + NW - 1) // NW

        def chunk_body(k, _, deg=deg, base=base, n_chunks=n_chunks,
                       last_r0=last_r0):
            cid = k * NW + wid

            @pl.when(cid < n_chunks)
            def _():
                r0 = jnp.minimum(cid * C, last_r0)
                pltpu.sync_copy(atoms.at[pl.ds(r0 + base, C)], self_v)
                if deg == 0:
                    pltpu.sync_copy(self_v, out.at[pl.ds(r0 + base, C)])
                else:
                    idx_v = idxs[deg - 1]
                    pltpu.sync_copy(adjs[deg - 1].at[pl.ds(r0, C)], idx_v)
                    pltpu.async_copy(atoms.at[idx_v],
                                     g_v.at[pl.ds(0, C * deg)], sem).wait()

                    def row_body(i, _):
                        for j in range(D // LANES):
                            sl = pl.ds(j * LANES, LANES)
                            acc = self_v[i, sl]
                            for n in range(deg):
                                acc = jnp.maximum(acc, g_v[i * deg + n, sl])
                            out_v[i, sl] = acc
                        return 0

                    lax.fori_loop(0, C, row_body, 0)
                    pltpu.sync_copy(out_v, out.at[pl.ds(r0 + base, C)])

            return 0

        lax.fori_loop(0, k_per_w, chunk_body, 0)


def kernel(atom_features, deg_slice, membership, n_samples,
           deg_adj_1, deg_adj_2, deg_adj_3, deg_adj_4, deg_adj_5,
           deg_adj_6, deg_adj_7, deg_adj_8, deg_adj_9, deg_adj_10):
    del deg_slice, membership, n_samples
    adjs = [a.reshape(PER_DEG, d + 1) for d, a in enumerate(
        (deg_adj_1, deg_adj_2, deg_adj_3, deg_adj_4, deg_adj_5,
         deg_adj_6, deg_adj_7, deg_adj_8, deg_adj_9, deg_adj_10))]
    mesh = plsc.VectorSubcoreMesh(core_axis_name="c", subcore_axis_name="s")
    run = pl.kernel(
        _body, mesh=mesh,
        out_type=jax.ShapeDtypeStruct((N, D), jnp.float32),
        scratch_types=(
            [pltpu.VMEM((C, d + 1), jnp.int32) for d in range(MAX_DEG)]
            + [
                pltpu.VMEM((C, D), jnp.float32),
                pltpu.VMEM((C * MAX_DEG, D), jnp.float32),
                pltpu.VMEM((C, D), jnp.float32),
                pltpu.SemaphoreType.DMA,
            ]
        ),
    )
    return run(atom_features, *adjs)


# speedup vs baseline: 3.1290x; 3.1290x over previous
import jax
import jax.numpy as jnp
from jax import lax
from jax.experimental import pallas as pl
from jax.experimental.pallas import tpu as pltpu
from jax.experimental.pallas import tpu_sc as plsc

N = 100000
D = 128
MAX_DEG = 10
PER_DEG = 9090
DEG0 = N - MAX_DEG * PER_DEG
STARTS = [0, DEG0] + [DEG0 + PER_DEG * k for k in range(1, MAX_DEG)]
SIZES = [DEG0] + [PER_DEG] * MAX_DEG

C = 64
NW = 32
LANES = 16
G = 64
NP = ((max(SIZES) + C - 1) // C) * C
N_CHUNKS = NP // C
K_PER_W = (N_CHUNKS + NW - 1) // NW


def _body(atoms, *rest):
    idx_hbm = rest[:MAX_DEG + 1]
    oidx_hbm = rest[MAX_DEG + 1:2 * (MAX_DEG + 1)]
    out = rest[2 * (MAX_DEG + 1)]
    idx_v, oidx_v, g_v, out_v, sem = rest[2 * (MAX_DEG + 1) + 1:]
    wid = lax.axis_index("s") * 2 + lax.axis_index("c")

    for deg in range(0, MAX_DEG + 1):
        w = deg + 1
        n_sub = (C * w) // G

        def chunk_body(k, _, deg=deg, w=w, n_sub=n_sub):
            cid = k * NW + wid

            @pl.when(cid < N_CHUNKS)
            def _():
                pltpu.sync_copy(idx_hbm[deg].at[pl.ds(cid * C * w, C * w)],
                                idx_v.at[pl.ds(0, C * w)])
                pltpu.sync_copy(oidx_hbm[deg].at[pl.ds(cid * C, C)], oidx_v)
                handles = [
                    pltpu.async_copy(atoms.at[idx_v.at[pl.ds(s * G, G)]],
                                     g_v.at[pl.ds(s * G, G)], sem)
                    for s in range(n_sub)
                ]
                for h in handles:
                    h.wait()
                if deg == 0:
                    pltpu.async_copy(g_v.at[pl.ds(0, C)],
                                     out.at[oidx_v], sem).wait()
                else:
                    def row_body(i, _):
                        for j in range(D // LANES):
                            sl = pl.ds(j * LANES, LANES)
                            acc = g_v[i * w, sl]
                            for nb in range(1, w):
                                acc = jnp.maximum(acc, g_v[i * w + nb, sl])
                            out_v[i, sl] = acc
                        return 0

                    lax.fori_loop(0, C, row_body, 0)
                    pltpu.async_copy(out_v, out.at[oidx_v], sem).wait()

            return 0

        lax.fori_loop(0, K_PER_W, chunk_body, 0)


def kernel(atom_features, deg_slice, membership, n_samples,
           deg_adj_1, deg_adj_2, deg_adj_3, deg_adj_4, deg_adj_5,
           deg_adj_6, deg_adj_7, deg_adj_8, deg_adj_9, deg_adj_10):
    del deg_slice, membership, n_samples
    adjs = [deg_adj_1, deg_adj_2, deg_adj_3, deg_adj_4, deg_adj_5,
            deg_adj_6, deg_adj_7, deg_adj_8, deg_adj_9, deg_adj_10]
    idx_lists = []
    oidx_lists = []
    for deg in range(0, MAX_DEG + 1):
        base = STARTS[deg]
        n_rows = SIZES[deg]
        row = jnp.arange(NP, dtype=jnp.int32)
        row = jnp.minimum(row, n_rows - 1)
        if deg == 0:
            aug = (base + row)[:, None]
        else:
            adj = adjs[deg - 1].reshape(PER_DEG, deg)
            aug = jnp.concatenate(
                [(base + row)[:, None], adj[row]], axis=1)
        idx_lists.append(aug.reshape(-1))
        oidx_lists.append(base + row)

    mesh = plsc.VectorSubcoreMesh(core_axis_name="c", subcore_axis_name="s")
    run = pl.kernel(
        _body, mesh=mesh,
        out_type=jax.ShapeDtypeStruct((N, D), jnp.float32),
        scratch_types=[
            pltpu.VMEM((C * (MAX_DEG + 1),), jnp.int32),
            pltpu.VMEM((C,), jnp.int32),
            pltpu.VMEM((C * (MAX_DEG + 1), D), jnp.float32),
            pltpu.VMEM((C, D), jnp.float32),
            pltpu.SemaphoreType.DMA,
        ],
    )
    return run(atom_features, *idx_lists, *oidx_lists)


# speedup vs baseline: 3.6738x; 1.1741x over previous
import jax
import jax.numpy as jnp
from jax import lax
from jax.experimental import pallas as pl
from jax.experimental.pallas import tpu as pltpu
from jax.experimental.pallas import tpu_sc as plsc

N = 100000
D = 128
MAX_DEG = 10
PER_DEG = 9090
DEG0 = N - MAX_DEG * PER_DEG
STARTS = [0, DEG0] + [DEG0 + PER_DEG * k for k in range(1, MAX_DEG)]
SIZES = [DEG0] + [PER_DEG] * MAX_DEG

NW = 32
LANES = 16

CHUNK_ROWS = [64, 64, 64, 64, 64, 48, 40, 40, 32, 32, 32]
SUBGATHERS = [[64], [128], [96, 96], [128, 128], [80] * 4, [96] * 3,
              [56] * 5, [80] * 4, [96] * 3, [80] * 4, [88] * 4]
CSIZES = sorted(set(CHUNK_ROWS))
GMAX = 352
N_CHUNKS = [-(-SIZES[deg] // CHUNK_ROWS[deg]) for deg in range(MAX_DEG + 1)]


def _chunk(deg, cid, atoms, idx_hbm, oidx_hbm, out, bufs):
    C = CHUNK_ROWS[deg]
    w = deg + 1
    idx_v, oidx_all, out_v, g_v, sem = bufs
    oidx_v = oidx_all[CSIZES.index(C)]

    def fire():
        pltpu.sync_copy(idx_hbm.at[pl.ds(cid * C * w, C * w)],
                        idx_v.at[pl.ds(0, C * w)])
        pltpu.sync_copy(oidx_hbm.at[pl.ds(cid * C, C)], oidx_v)
        handles = []
        off = 0
        for g in SUBGATHERS[deg]:
            handles.append(
                pltpu.async_copy(atoms.at[idx_v.at[pl.ds(off, g)]],
                                 g_v.at[pl.ds(off, g)], sem))
            off += g
        return handles

    def finish(handles):
        for h in handles:
            h.wait()
        if deg == 0:
            pltpu.async_copy(g_v.at[pl.ds(0, C)], out.at[oidx_v], sem).wait()
            return

        def row_body(i, _):
            for j in range(D // LANES):
                sl = pl.ds(j * LANES, LANES)
                acc = g_v[i * w, sl]
                for nb in range(1, w):
                    acc = jnp.maximum(acc, g_v[i * w + nb, sl])
                out_v[i, sl] = acc
            return 0

        lax.fori_loop(0, C, row_body, 0)
        pltpu.async_copy(out_v.at[pl.ds(0, C)], out.at[oidx_v], sem).wait()

    return fire, finish


def _body(atoms, *rest):
    nd = MAX_DEG + 1
    idx_hbm = rest[:nd]
    oidx_hbm = rest[nd:2 * nd]
    out = rest[2 * nd]
    (idx_a, idx_b, o32_a, o32_b, o40_a, o40_b, o48_a, o48_b, o64_a, o64_b,
     out_a, out_b, g_a, g_b, sem_a, sem_b) = rest[2 * nd + 1:]
    bufs_a = (idx_a, [o32_a, o40_a, o48_a, o64_a], out_a, g_a, sem_a)
    bufs_b = (idx_b, [o32_b, o40_b, o48_b, o64_b], out_b, g_b, sem_b)
    wid = lax.axis_index("s") * 2 + lax.axis_index("c")

    for deg in range(0, MAX_DEG + 1):
        n_chunks = N_CHUNKS[deg]
        k_max = -(-n_chunks // NW)
        m_max = -(-k_max // 2)

        def pair_body(m, _, deg=deg, n_chunks=n_chunks):
            cid_a = (2 * m) * NW + wid
            cid_b = (2 * m + 1) * NW + wid
            fire_a, finish_a = _chunk(deg, cid_a, atoms, idx_hbm[deg],
                                      oidx_hbm[deg], out, bufs_a)
            fire_b, finish_b = _chunk(deg, cid_b, atoms, idx_hbm[deg],
                                      oidx_hbm[deg], out, bufs_b)

            @pl.when(cid_a < n_chunks)
            def _():
                ha = fire_a()

                @pl.when(cid_b < n_chunks)
                def _():
                    hb = fire_b()
                    finish_a(ha)
                    finish_b(hb)

                @pl.when(cid_b >= n_chunks)
                def _():
                    finish_a(ha)

            return 0

        lax.fori_loop(0, m_max, pair_body, 0)


def kernel(atom_features, deg_slice, membership, n_samples,
           deg_adj_1, deg_adj_2, deg_adj_3, deg_adj_4, deg_adj_5,
           deg_adj_6, deg_adj_7, deg_adj_8, deg_adj_9, deg_adj_10):
    del deg_slice, membership, n_samples
    adjs = [deg_adj_1, deg_adj_2, deg_adj_3, deg_adj_4, deg_adj_5,
            deg_adj_6, deg_adj_7, deg_adj_8, deg_adj_9, deg_adj_10]
    idx_lists = []
    oidx_lists = []
    for deg in range(0, MAX_DEG + 1):
        base = STARTS[deg]
        n_rows = SIZES[deg]
        np_rows = N_CHUNKS[deg] * CHUNK_ROWS[deg]
        row = jnp.minimum(jnp.arange(np_rows, dtype=jnp.int32), n_rows - 1)
        if deg == 0:
            aug = (base + row)[:, None]
        else:
            adj = adjs[deg - 1].reshape(PER_DEG, deg)
            aug = jnp.concatenate([(base + row)[:, None], adj[row]], axis=1)
        idx_lists.append(aug.reshape(-1))
        oidx_lists.append(base + row)

    mesh = plsc.VectorSubcoreMesh(core_axis_name="c", subcore_axis_name="s")
    run = pl.kernel(
        _body, mesh=mesh,
        out_type=jax.ShapeDtypeStruct((N, D), jnp.float32),
        scratch_types=(
            [pltpu.VMEM((GMAX,), jnp.int32)] * 2
            + [pltpu.VMEM((c,), jnp.int32)
               for c in CSIZES for _ in (0, 1)]
            + [pltpu.VMEM((max(CHUNK_ROWS), D), jnp.float32)] * 2
            + [pltpu.VMEM((GMAX, D), jnp.float32)] * 2
            + [pltpu.SemaphoreType.DMA] * 2
        ),
    )
    return run(atom_features, *idx_lists, *oidx_lists)


# speedup vs baseline: 4.8987x; 1.3334x over previous
import jax
import jax.numpy as jnp
from jax import lax
from jax.experimental import pallas as pl
from jax.experimental.pallas import tpu as pltpu
from jax.experimental.pallas import tpu_sc as plsc

N = 100000
D = 128
MAX_DEG = 10
PER_DEG = 9090
DEG0 = N - MAX_DEG * PER_DEG
STARTS = [0, DEG0] + [DEG0 + PER_DEG * k for k in range(1, MAX_DEG)]
SIZES = [DEG0] + [PER_DEG] * MAX_DEG

NW = 32
LANES = 16

CHUNK_ROWS = [64, 64, 64, 64, 64, 64, 48, 40, 40, 32, 32]
SUBGATHERS = [None, [64], [128], [96, 96], [128, 128], [80] * 4,
              [96] * 3, [56] * 5, [80] * 4, [96] * 3, [80] * 4]
CSIZES = sorted(set(CHUNK_ROWS))
GMAX = 320
CMAX = max(CHUNK_ROWS)
N_CHUNKS = [-(-SIZES[deg] // CHUNK_ROWS[deg]) for deg in range(MAX_DEG + 1)]


def _chunk(deg, cid, atoms, adj_flat, oidx_hbm, out, bufs):
    C = CHUNK_ROWS[deg]
    d = deg
    sidx_all, idx_v, self_v, out_v, g_v, sem = bufs
    sidx_v = sidx_all[CSIZES.index(C)]

    def fire():
        pltpu.sync_copy(oidx_hbm.at[pl.ds(cid * C, C)], sidx_v)
        handles = [pltpu.async_copy(atoms.at[sidx_v],
                                    self_v.at[pl.ds(0, C)], sem)]
        if d > 0:
            pltpu.sync_copy(adj_flat.at[pl.ds(cid * C * d, C * d)],
                            idx_v.at[pl.ds(0, C * d)])
            off = 0
            for g in SUBGATHERS[deg]:
                handles.append(
                    pltpu.async_copy(atoms.at[idx_v.at[pl.ds(off, g)]],
                                     g_v.at[pl.ds(off, g)], sem))
                off += g
        return handles

    def finish(handles):
        for h in handles:
            h.wait()
        if d == 0:
            return pltpu.async_copy(self_v.at[pl.ds(0, C)],
                                    out.at[sidx_v], sem)

        def row_body(i, _):
            for j in range(D // LANES):
                sl = pl.ds(j * LANES, LANES)
                acc = self_v[i, sl]
                for nb in range(d):
                    acc = jnp.maximum(acc, g_v[i * d + nb, sl])
                out_v[i, sl] = acc
            return 0

        lax.fori_loop(0, C, row_body, 0)
        return pltpu.async_copy(out_v.at[pl.ds(0, C)], out.at[sidx_v], sem)

    return fire, finish


def _body(atoms, a1, a2, a3, a4, a5, a6, a7, a8, a9, a10,
          o0, o1, o2, o3, o4, o5, o6, o7, o8, o9, o10, out, *scratch):
    adj_flats = [None, a1, a2, a3, a4, a5, a6, a7, a8, a9, a10]
    oidx_hbms = [o0, o1, o2, o3, o4, o5, o6, o7, o8, o9, o10]
    ns = len(CSIZES)
    sidx = scratch[:2 * ns]
    (idx_a, idx_b, self_a, self_b, out_a, out_b, g_a, g_b,
     sem_a, sem_b) = scratch[2 * ns:]
    bufs_a = (sidx[0::2], idx_a, self_a, out_a, g_a, sem_a)
    bufs_b = (sidx[1::2], idx_b, self_b, out_b, g_b, sem_b)
    wid = lax.axis_index("s") * 2 + lax.axis_index("c")

    for deg in range(0, MAX_DEG + 1):
        n_chunks = N_CHUNKS[deg]
        k_max = -(-n_chunks // NW)
        m_max = -(-k_max // 2)

        def pair_body(m, _, deg=deg, n_chunks=n_chunks):
            cid_a = (2 * m) * NW + wid
            cid_b = (2 * m + 1) * NW + wid
            fire_a, finish_a = _chunk(deg, cid_a, atoms, adj_flats[deg],
                                      oidx_hbms[deg], out, bufs_a)
            fire_b, finish_b = _chunk(deg, cid_b, atoms, adj_flats[deg],
                                      oidx_hbms[deg], out, bufs_b)

            @pl.when(cid_a < n_chunks)
            def _():
                ha = fire_a()

                @pl.when(cid_b < n_chunks)
                def _():
                    hb = fire_b()
                    sa = finish_a(ha)
                    sb = finish_b(hb)
                    sa.wait()
                    sb.wait()

                @pl.when(cid_b >= n_chunks)
                def _():
                    finish_a(ha).wait()

            return 0

        lax.fori_loop(0, m_max, pair_body, 0)


def kernel(atom_features, deg_slice, membership, n_samples,
           deg_adj_1, deg_adj_2, deg_adj_3, deg_adj_4, deg_adj_5,
           deg_adj_6, deg_adj_7, deg_adj_8, deg_adj_9, deg_adj_10):
    del deg_slice, membership, n_samples
    adjs = [deg_adj_1, deg_adj_2, deg_adj_3, deg_adj_4, deg_adj_5,
            deg_adj_6, deg_adj_7, deg_adj_8, deg_adj_9, deg_adj_10]
    flats = []
    for d, a in enumerate(adjs, start=1):
        C = CHUNK_ROWS[d]
        n_full = PER_DEG // C
        a2 = a.reshape(PER_DEG, d)
        flat = a2.reshape(-1)
        if PER_DEG % C:
            flat = jnp.concatenate(
                [flat[: n_full * C * d], a2[PER_DEG - C:].reshape(-1)])
        flats.append(flat)

    oidxs = []
    for deg in range(0, MAX_DEG + 1):
        C = CHUNK_ROWS[deg]
        n_rows = SIZES[deg]
        np_rows = N_CHUNKS[deg] * C
        k = jnp.arange(np_rows, dtype=jnp.int32) // C
        r0 = jnp.minimum(k * C, n_rows - C)
        oidxs.append(STARTS[deg] + r0
                     + jnp.arange(np_rows, dtype=jnp.int32) % C)

    mesh = plsc.VectorSubcoreMesh(core_axis_name="c", subcore_axis_name="s")
    run = pl.kernel(
        _body, mesh=mesh,
        out_type=jax.ShapeDtypeStruct((N, D), jnp.float32),
        scratch_types=(
            [pltpu.VMEM((c,), jnp.int32)
             for c in CSIZES for _ in (0, 1)]
            + [pltpu.VMEM((GMAX,), jnp.int32)] * 2
            + [pltpu.VMEM((CMAX, D), jnp.float32)] * 2
            + [pltpu.VMEM((CMAX, D), jnp.float32)] * 2
            + [pltpu.VMEM((GMAX, D), jnp.float32)] * 2
            + [pltpu.SemaphoreType.DMA] * 2
        ),
    )
    return run(atom_features, *flats, *oidxs)


# speedup vs baseline: 5.3548x; 1.0931x over previous
import jax
import jax.numpy as jnp
from jax import lax
from jax.experimental import pallas as pl
from jax.experimental.pallas import tpu as pltpu
from jax.experimental.pallas import tpu_sc as plsc

N = 100000
D = 128
MAX_DEG = 10
PER_DEG = 9090
DEG0 = N - MAX_DEG * PER_DEG
STARTS = [0, DEG0] + [DEG0 + PER_DEG * k for k in range(1, MAX_DEG)]
SIZES = [DEG0] + [PER_DEG] * MAX_DEG

NW = 32
LANES = 16

CHUNK_ROWS = [64, 64, 64, 64, 64, 64, 48, 40, 40, 32, 32]
SUBGATHERS = [None, [64], [128], [96, 96], [128, 128], [80] * 4,
              [96] * 3, [56] * 5, [80] * 4, [96] * 3, [80] * 4]
CSIZES = sorted(set(CHUNK_ROWS))
GMAX = 320
CMAX = max(CHUNK_ROWS)
N_CHUNKS = [-(-SIZES[deg] // CHUNK_ROWS[deg]) for deg in range(MAX_DEG + 1)]


class _Set:
    def __init__(self, deg, atoms, adj_flat, oidx_hbm, out, bufs):
        self.deg = deg
        self.C = CHUNK_ROWS[deg]
        self.atoms, self.adj, self.oidx_hbm, self.out = (
            atoms, adj_flat, oidx_hbm, out)
        si = CSIZES.index(self.C)
        self.sidx_v = bufs[si]
        self.oidx_v = bufs[4 + si]
        (self.idx_v, self.self_v, self.out_v, self.g_v,
         self.sem_g, self.sem_s) = bufs[8:]

    def fire(self, cid, rot):
        C, d = self.C, self.deg
        pltpu.sync_copy(self.oidx_hbm.at[pl.ds(cid * C, C)], self.sidx_v)
        pltpu.sync_copy(self.oidx_hbm.at[pl.ds(cid * C, C)],
                        self.oidx_v.at[rot])
        pltpu.async_copy(self.atoms.at[self.sidx_v],
                         self.self_v.at[pl.ds(0, C)], self.sem_g)
        if d > 0:
            pltpu.sync_copy(self.adj.at[pl.ds(cid * C * d, C * d)],
                            self.idx_v.at[pl.ds(0, C * d)])
            off = 0
            for g in SUBGATHERS[d]:
                pltpu.async_copy(
                    self.atoms.at[self.idx_v.at[pl.ds(off, g)]],
                    self.g_v.at[pl.ds(off, g)], self.sem_g)
                off += g

    def wait_gathers(self):
        C, d = self.C, self.deg
        pltpu.make_async_copy(self.atoms.at[self.sidx_v],
                              self.self_v.at[pl.ds(0, C)],
                              self.sem_g).wait()
        if d > 0:
            off = 0
            for g in SUBGATHERS[d]:
                pltpu.make_async_copy(
                    self.atoms.at[self.idx_v.at[pl.ds(off, g)]],
                    self.g_v.at[pl.ds(off, g)], self.sem_g).wait()
                off += g

    def _scatter_src(self):
        return self.out_v.at[pl.ds(0, self.C)]

    def compute_scatter(self, rot):
        C, d = self.C, self.deg
        self_v, out_v, g_v = self.self_v, self.out_v, self.g_v

        def row_body(i, _):
            for j in range(D // LANES):
                sl = pl.ds(j * LANES, LANES)
                acc = self_v[i, sl]
                for nb in range(d):
                    acc = jnp.maximum(acc, g_v[i * d + nb, sl])
                out_v[i, sl] = acc
            return 0

        lax.fori_loop(0, C, row_body, 0)
        pltpu.async_copy(self._scatter_src(), self.out.at[self.oidx_v.at[rot]],
                         self.sem_s)

    def wait_scatter(self):
        pltpu.make_async_copy(self._scatter_src(),
                              self.out.at[self.oidx_v.at[0]],
                              self.sem_s).wait()


def _body(atoms, a1, a2, a3, a4, a5, a6, a7, a8, a9, a10,
          o0, o1, o2, o3, o4, o5, o6, o7, o8, o9, o10, out, *scratch):
    adj_flats = [None, a1, a2, a3, a4, a5, a6, a7, a8, a9, a10]
    oidx_hbms = [o0, o1, o2, o3, o4, o5, o6, o7, o8, o9, o10]
    half = len(scratch) // 2
    sa, sb = scratch[:half], scratch[half:]
    wid = lax.axis_index("s") * 2 + lax.axis_index("c")

    for deg in range(0, MAX_DEG + 1):
        n = N_CHUNKS[deg]
        k_max = -(-n // NW)
        m_max = -(-k_max // 2)
        A = _Set(deg, atoms, adj_flats[deg], oidx_hbms[deg], out, sa)
        B = _Set(deg, atoms, adj_flats[deg], oidx_hbms[deg], out, sb)

        @pl.when(wid < n)
        def _(A=A):
            A.fire(wid, 0)

        def pipe_body(m, _, A=A, B=B, n=n):
            cid0 = (2 * m) * NW + wid
            cid1 = cid0 + NW
            cid2 = cid0 + 2 * NW
            rot = jnp.bitwise_and(m, 1)

            @pl.when(cid1 < n)
            def _():
                B.fire(cid1, rot)

            @pl.when(cid0 < n)
            def _():
                A.wait_gathers()

                @pl.when(m >= 1)
                def _():
                    A.wait_scatter()

                A.compute_scatter(rot)

            @pl.when(cid2 < n)
            def _():
                A.fire(cid2, 1 - rot)

            @pl.when(cid1 < n)
            def _():
                B.wait_gathers()

                @pl.when(m >= 1)
                def _():
                    B.wait_scatter()

                B.compute_scatter(rot)

            return 0

        lax.fori_loop(0, m_max, pipe_body, 0)

        for j in range(2 * m_max):
            cidj = j * NW + wid
            S = A if j % 2 == 0 else B

            @pl.when(jnp.logical_and(cidj < n, cidj + 2 * NW >= n))
            def _(S=S):
                S.wait_scatter()


def kernel(atom_features, deg_slice, membership, n_samples,
           deg_adj_1, deg_adj_2, deg_adj_3, deg_adj_4, deg_adj_5,
           deg_adj_6, deg_adj_7, deg_adj_8, deg_adj_9, deg_adj_10):
    del deg_slice, membership, n_samples
    adjs = [deg_adj_1, deg_adj_2, deg_adj_3, deg_adj_4, deg_adj_5,
            deg_adj_6, deg_adj_7, deg_adj_8, deg_adj_9, deg_adj_10]
    flats = []
    for d, a in enumerate(adjs, start=1):
        C = CHUNK_ROWS[d]
        n_full = PER_DEG // C
        a2 = a.reshape(PER_DEG, d)
        flat = a2.reshape(-1)
        if PER_DEG % C:
            flat = jnp.concatenate(
                [flat[: n_full * C * d], a2[PER_DEG - C:].reshape(-1)])
        flats.append(flat)

    oidxs = []
    for deg in range(0, MAX_DEG + 1):
        C = CHUNK_ROWS[deg]
        n_rows = SIZES[deg]
        np_rows = N_CHUNKS[deg] * C
        k = jnp.arange(np_rows, dtype=jnp.int32) // C
        r0 = jnp.minimum(k * C, n_rows - C)
        oidxs.append(STARTS[deg] + r0
                     + jnp.arange(np_rows, dtype=jnp.int32) % C)

    def one_set():
        return ([pltpu.VMEM((c,), jnp.int32) for c in CSIZES]
                + [pltpu.VMEM((2, c), jnp.int32) for c in CSIZES]
                + [pltpu.VMEM((GMAX,), jnp.int32),
                   pltpu.VMEM((CMAX, D), jnp.float32),
                   pltpu.VMEM((CMAX, D), jnp.float32),
                   pltpu.VMEM((GMAX, D), jnp.float32),
                   pltpu.SemaphoreType.DMA,
                   pltpu.SemaphoreType.DMA])

    mesh = plsc.VectorSubcoreMesh(core_axis_name="c", subcore_axis_name="s")
    run = pl.kernel(
        _body, mesh=mesh,
        out_type=jax.ShapeDtypeStruct((N, D), jnp.float32),
        scratch_types=one_set() + one_set(),
    )
    return run(atom_features, *flats, *oidxs)


# speedup vs baseline: 6.1389x; 1.1464x over previous
import jax
import jax.numpy as jnp
from jax import lax
from jax.experimental import pallas as pl
from jax.experimental.pallas import tpu as pltpu
from jax.experimental.pallas import tpu_sc as plsc

N = 100000
D = 128
MAX_DEG = 10
PER_DEG = 9090
DEG0 = N - MAX_DEG * PER_DEG
STARTS = [0, DEG0] + [DEG0 + PER_DEG * k for k in range(1, MAX_DEG)]
SIZES = [DEG0] + [PER_DEG] * MAX_DEG

NW = 32
LANES = 16

CHUNK_ROWS = [64, 64, 64, 64, 64, 64, 48, 40, 40, 32, 32]
SUBGATHERS = [None, [64], [128], [96, 96], [128, 128], [80] * 4,
              [96] * 3, [56] * 5, [80] * 4, [96] * 3, [80] * 4]
GMAX = 320
CMAX = max(CHUNK_ROWS)
N_CHUNKS = [-(-SIZES[deg] // CHUNK_ROWS[deg]) for deg in range(MAX_DEG + 1)]
K_MAX = [-(-n // NW) for n in N_CHUNKS]
IMAX = max(K_MAX[deg] * CHUNK_ROWS[deg] * max(deg, 1)
           for deg in range(MAX_DEG + 1))
SMAX = max(K_MAX[deg] * CHUNK_ROWS[deg] for deg in range(MAX_DEG + 1))


class _Deg:
    def __init__(self, deg, atoms, adj_flat, oidx_hbm, out, span, sets):
        self.deg = deg
        self.C = CHUNK_ROWS[deg]
        self.atoms, self.adj, self.oidx_hbm, self.out = (
            atoms, adj_flat, oidx_hbm, out)
        self.idx_v, self.sidx_v = span
        self.sets = sets

    def prefetch(self, lo):
        C, d, K = self.C, self.deg, K_MAX[self.deg]
        pltpu.sync_copy(self.oidx_hbm.at[pl.ds(lo * C, K * C)],
                        self.sidx_v.at[pl.ds(0, K * C)])
        if d > 0:
            pltpu.sync_copy(self.adj.at[pl.ds(lo * C * d, K * C * d)],
                            self.idx_v.at[pl.ds(0, K * C * d)])

    def _gathers(self, j, set_i):
        C, d = self.C, self.deg
        self_v, out_v, g_v, sem_g, sem_s = self.sets[set_i]
        copies = [(self.atoms.at[self.sidx_v.at[pl.ds(j * C, C)]],
                   self_v.at[pl.ds(0, C)], sem_g)]
        if d > 0:
            off = 0
            for g in SUBGATHERS[d]:
                copies.append(
                    (self.atoms.at[self.idx_v.at[pl.ds(j * C * d + off, g)]],
                     g_v.at[pl.ds(off, g)], sem_g))
                off += g
        return copies

    def fire(self, j, set_i):
        for src, dst, sem in self._gathers(j, set_i):
            pltpu.async_copy(src, dst, sem)

    def process(self, j, set_i, first):
        C, d = self.C, self.deg
        self_v, out_v, g_v, sem_g, sem_s = self.sets[set_i]
        for src, dst, sem in self._gathers(j, set_i):
            pltpu.make_async_copy(src, dst, sem).wait()

        @pl.when(jnp.logical_not(first))
        def _():
            self.wait_scatter(set_i)

        def row_body(i, _):
            for jj in range(D // LANES):
                sl = pl.ds(jj * LANES, LANES)
                acc = self_v[i, sl]
                for nb in range(d):
                    acc = jnp.maximum(acc, g_v[i * d + nb, sl])
                out_v[i, sl] = acc
            return 0

        lax.fori_loop(0, C, row_body, 0)
        pltpu.async_copy(out_v.at[pl.ds(0, C)],
                         self.out.at[self.sidx_v.at[pl.ds(j * C, C)]], sem_s)

    def wait_scatter(self, set_i):
        C = self.C
        self_v, out_v, g_v, sem_g, sem_s = self.sets[set_i]
        pltpu.make_async_copy(out_v.at[pl.ds(0, C)],
                              self.out.at[self.sidx_v.at[pl.ds(0, C)]],
                              sem_s).wait()


def _body(atoms, a1, a2, a3, a4, a5, a6, a7, a8, a9, a10,
          o0, o1, o2, o3, o4, o5, o6, o7, o8, o9, o10, out,
          idx_v, sidx_v, self_a, out_a, g_a, self_b, out_b, g_b,
          sem_ga, sem_sa, sem_gb, sem_sb):
    adj_flats = [None, a1, a2, a3, a4, a5, a6, a7, a8, a9, a10]
    oidx_hbms = [o0, o1, o2, o3, o4, o5, o6, o7, o8, o9, o10]
    span = (idx_v, sidx_v)
    sets = [(self_a, out_a, g_a, sem_ga, sem_sa),
            (self_b, out_b, g_b, sem_gb, sem_sb)]
    wid = lax.axis_index("s") * 2 + lax.axis_index("c")

    for deg in range(0, MAX_DEG + 1):
        n = N_CHUNKS[deg]
        m_max = -(-K_MAX[deg] // 2)
        dd = _Deg(deg, atoms, adj_flats[deg], oidx_hbms[deg], out, span,
                  sets)
        lo = (wid * n) >> 5
        kw = (((wid + 1) * n) >> 5) - lo

        dd.prefetch(lo)

        @pl.when(kw > 0)
        def _(dd=dd):
            dd.fire(0, 0)

        def pipe_body(m, _, dd=dd, kw=kw):
            j0 = 2 * m
            j1 = j0 + 1
            j2 = j0 + 2

            @pl.when(j1 < kw)
            def _():
                dd.fire(j1, 1)

            @pl.when(j0 < kw)
            def _():
                dd.process(j0, 0, m < 1)

            @pl.when(j2 < kw)
            def _():
                dd.fire(j2, 0)

            @pl.when(j1 < kw)
            def _():
                dd.process(j1, 1, m < 1)

            return 0

        lax.fori_loop(0, m_max, pipe_body, 0)

        for j in range(2 * m_max):

            @pl.when(jnp.logical_and(j < kw, j + 2 >= kw))
            def _(dd=dd, j=j):
                dd.wait_scatter(j % 2)


def kernel(atom_features, deg_slice, membership, n_samples,
           deg_adj_1, deg_adj_2, deg_adj_3, deg_adj_4, deg_adj_5,
           deg_adj_6, deg_adj_7, deg_adj_8, deg_adj_9, deg_adj_10):
    del deg_slice, membership, n_samples
    adjs = [deg_adj_1, deg_adj_2, deg_adj_3, deg_adj_4, deg_adj_5,
            deg_adj_6, deg_adj_7, deg_adj_8, deg_adj_9, deg_adj_10]
    flats = []
    for d, a in enumerate(adjs, start=1):
        C = CHUNK_ROWS[d]
        n_full = PER_DEG // C
        a2 = a.reshape(PER_DEG, d)
        flat = a2.reshape(-1)
        if PER_DEG % C:
            flat = jnp.concatenate(
                [flat[: n_full * C * d], a2[PER_DEG - C:].reshape(-1)])
        flats.append(flat)

    oidxs = []
    for deg in range(0, MAX_DEG + 1):
        C = CHUNK_ROWS[deg]
        n_rows = SIZES[deg]
        np_rows = N_CHUNKS[deg] * C
        k = jnp.arange(np_rows, dtype=jnp.int32) // C
        r0 = jnp.minimum(k * C, n_rows - C)
        oidxs.append(STARTS[deg] + r0
                     + jnp.arange(np_rows, dtype=jnp.int32) % C)

    mesh = plsc.VectorSubcoreMesh(core_axis_name="c", subcore_axis_name="s")
    run = pl.kernel(
        _body, mesh=mesh,
        out_type=jax.ShapeDtypeStruct((N, D), jnp.float32),
        scratch_types=[
            pltpu.VMEM((IMAX,), jnp.int32),
            pltpu.VMEM((SMAX,), jnp.int32),
            pltpu.VMEM((CMAX, D), jnp.float32),
            pltpu.VMEM((CMAX, D), jnp.float32),
            pltpu.VMEM((GMAX, D), jnp.float32),
            pltpu.VMEM((CMAX, D), jnp.float32),
            pltpu.VMEM((CMAX, D), jnp.float32),
            pltpu.VMEM((GMAX, D), jnp.float32),
            pltpu.SemaphoreType.DMA,
            pltpu.SemaphoreType.DMA,
            pltpu.SemaphoreType.DMA,
            pltpu.SemaphoreType.DMA,
        ],
    )
    return run(atom_features, *flats, *oidxs)
